# baseline (speedup 1.0000x reference)
import jax
import jax.numpy as jnp
from jax import lax
from jax.experimental import pallas as pl
from jax.experimental.pallas import tpu as pltpu

B = 128
T = 32
P = 2048
D = 64
NB = 4
K = 2
TEMP = 8.0
E = T * NB
CHUNK = 32
HALF = P / 2.0


def _ring_kernel(xT_ref, in_Wt_ref, in_b_ref, out_Wt_ref, out_b_ref,
                 proc_Wt_ref, proc_b_ref, destT_ref, jump_Wc_ref, jump_b_ref,
                 cs_ref, pb_ref, ptr0_ref,
                 y_ref,
                 SU, W5, BASE, PTR, HID):
    L = proc_Wt_ref.shape[0]
    PTR[...] = ptr0_ref[...]
    HID[...] = jnp.zeros_like(HID)
    SU[...] = jnp.zeros_like(SU)
    W5[...] = jnp.zeros_like(W5)
    BASE[...] = jnp.zeros_like(BASE)

    offs5 = lax.broadcasted_iota(jnp.int32, (5, B), 0).astype(jnp.float32) - K
    iotaC = lax.broadcasted_iota(jnp.int32, (CHUNK, 1), 0).astype(jnp.float32)
    iotaP = lax.broadcasted_iota(jnp.int32, (P, B), 0)

    def make_step(k):
        def step_t(t, _):
            xt = xT_ref[t]
            inp = jnp.dot(in_Wt_ref[...], xt,
                          preferred_element_type=jnp.float32) + in_b_ref[...]
            for bi in range(NB):
                ptr = PTR[bi][None, :]
                base_i = jnp.clip(jnp.floor(ptr).astype(jnp.int32), 0, P - 1)
                base_f = base_i.astype(jnp.float32)
                idx_f = jnp.mod(base_f + offs5, P)
                delta = jnp.remainder(idx_f - ptr + HALF, P) - HALF
                logits = -(delta * delta) / TEMP
                mx = jnp.max(logits, axis=0, keepdims=True)
                ex = jnp.exp(logits - mx)
                w = ex / jnp.sum(ex, axis=0, keepdims=True)

                context = jnp.zeros((D, B), jnp.float32)
                for ci in range(k + 1):
                    sl = slice(CHUNK * ci, CHUNK * (ci + 1))
                    dd = jnp.remainder(base_f - BASE[sl] + HALF, P) - HALF
                    c = jnp.zeros((CHUNK, B), jnp.float32)
                    for jp in range(5):
                        g = jnp.zeros((CHUNK, B), jnp.float32)
                        for m in range(5):
                            g = g + jnp.where(dd == float(jp - m),
                                              w[m][None, :], 0.0)
                        c = c + W5[jp, sl] * g
                    if ci == k:
                        s_rel = (t * NB + bi - CHUNK * k).astype(jnp.float32)
                        c = c * jnp.where(iotaC < s_rel, 1.0, 0.0)
                    context = context + jnp.sum(c[:, None, :] * SU[sl],
                                                axis=0)

                comb = inp + cs_ref[bi] * context + 0.1 * pb_ref[bi]
                su = jnp.tanh(comb + HID[bi])
                for l in range(L):
                    su = jnp.tanh(jnp.dot(proc_Wt_ref[l], su,
                                          preferred_element_type=jnp.float32)
                                  + proc_b_ref[l])
                HID[bi] = su
                SU[pl.ds(t * NB + bi, 1)] = su[None]
                W5[:, pl.ds(t * NB + bi, 1), :] = w[:, None, :]
                BASE[pl.ds(t * NB + bi, 1)] = base_f
                if bi == 0:
                    ACC = su
                else:
                    ACC = ACC + su

                jl = jnp.sum(jump_Wc_ref[bi] * su, axis=0, keepdims=True) \
                    + jump_b_ref[bi]
                jd = jnp.where(jax.nn.sigmoid(jl) > 0.5, 1.0, 0.0)
                walk = jnp.remainder(ptr + 1.0, P)
                onehot = iotaP == base_i
                destv = jnp.sum(jnp.where(onehot, destT_ref[:, bi:bi + 1],
                                          0.0),
                                axis=0, keepdims=True)
                PTR[bi] = jnp.remainder(jd * destv + (1.0 - jd) * walk, P)[0]
            y_ref[pl.ds(t, 1)] = (jnp.dot(out_Wt_ref[...], ACC * (1.0 / NB),
                                          preferred_element_type=jnp.float32)
                                  + out_b_ref[...])[None]
            return 0
        return step_t

    for k in range(4):
        lax.fori_loop(8 * k, 8 * (k + 1), make_step(k), 0)


@jax.jit
def kernel(x, in_W, in_b, out_W, out_b, proc_W, proc_b, dest, jump_W, jump_b,
           ctx, phase, ptr_init):
    xT = jnp.transpose(x, (1, 2, 0))
    in_Wt = jnp.transpose(in_W)
    out_Wt = jnp.transpose(out_W)
    proc_Wt = jnp.transpose(proc_W, (0, 2, 1))
    destT = jnp.transpose(dest)
    pb = jnp.concatenate(
        [phase, jnp.zeros((NB, D - phase.shape[1]), phase.dtype)], axis=1)
    yT = pl.pallas_call(
        _ring_kernel,
        out_shape=jax.ShapeDtypeStruct((T, 8, B), jnp.float32),
        scratch_shapes=[
            pltpu.VMEM((E, D, B), jnp.float32),
            pltpu.VMEM((5, E, B), jnp.float32),
            pltpu.VMEM((E, B), jnp.float32),
            pltpu.VMEM((NB, B), jnp.float32),
            pltpu.VMEM((NB, D, B), jnp.float32),
        ],
    )(xT, in_Wt, in_b[:, None], out_Wt, out_b[:, None],
      proc_Wt, proc_b[:, :, None], destT, jump_W[:, :, None],
      jump_b[:, None, None], jax.nn.sigmoid(ctx)[:, None, None],
      pb[:, :, None], ptr_init)
    return jnp.transpose(yT, (2, 0, 1))


# speedup vs baseline: 1.1377x; 1.1377x over previous
import jax
import jax.numpy as jnp
from jax import lax
from jax.experimental import pallas as pl
from jax.experimental.pallas import tpu as pltpu

B = 128
T = 32
P = 2048
D = 64
NB = 4
K = 2
TEMP = 8.0
E = T * NB
CHUNK = 32
HALF = P / 2.0


def _ring_kernel(xT_ref, in_Wt_ref, in_b_ref, out_Wt_ref, out_b_ref,
                 proc_Wt_ref, proc_b_ref, destT_ref, jump_Wc_ref, jump_b_ref,
                 cs_ref, pb_ref, ptr0_ref,
                 y_ref,
                 SU, W5, BASE, PTR, HID):
    L = proc_Wt_ref.shape[0]
    PTR[...] = ptr0_ref[...]
    HID[...] = jnp.zeros_like(HID)
    SU[...] = jnp.zeros_like(SU)
    W5[...] = jnp.zeros_like(W5)
    BASE[...] = jnp.zeros_like(BASE)

    offs5 = lax.broadcasted_iota(jnp.int32, (5, B), 0).astype(jnp.float32) - K
    iotaC = lax.broadcasted_iota(jnp.int32, (CHUNK, 1), 0).astype(jnp.float32)
    iotaP = lax.broadcasted_iota(jnp.int32, (P, B), 0)

    def make_step(k):
        def step_t(t, _):
            xt = xT_ref[t]
            inp = jnp.dot(in_Wt_ref[...], xt,
                          preferred_element_type=jnp.float32) + in_b_ref[...]
            for bi in range(NB):
                ptr = PTR[bi][None, :]
                base_i = jnp.clip(jnp.floor(ptr).astype(jnp.int32), 0, P - 1)
                base_f = base_i.astype(jnp.float32)
                idx_f = jnp.mod(base_f + offs5, P)
                delta = jnp.remainder(idx_f - ptr + HALF, P) - HALF
                logits = -(delta * delta) / TEMP
                mx = jnp.max(logits, axis=0, keepdims=True)
                ex = jnp.exp(logits - mx)
                w = ex / jnp.sum(ex, axis=0, keepdims=True)

                context = jnp.zeros((D, B), jnp.float32)
                for ci in range(k + 1):
                    sl = slice(CHUNK * ci, CHUNK * (ci + 1))
                    dd = jnp.remainder(base_f - BASE[sl] + HALF, P) - HALF
                    c = jnp.zeros((CHUNK, B), jnp.float32)
                    for jp in range(5):
                        g = jnp.zeros((CHUNK, B), jnp.float32)
                        for m in range(5):
                            g = g + jnp.where(dd == float(jp - m),
                                              w[m][None, :], 0.0)
                        c = c + W5[jp, sl] * g
                    if ci == k:
                        s_rel = (t * NB + bi - CHUNK * k).astype(jnp.float32)
                        c = c * jnp.where(iotaC < s_rel, 1.0, 0.0)
                    for e in range(CHUNK):
                        context = context + c[e][None, :] * SU[CHUNK * ci + e]

                comb = inp + cs_ref[bi] * context + 0.1 * pb_ref[bi]
                su = jnp.tanh(comb + HID[bi])
                for l in range(L):
                    su = jnp.tanh(jnp.dot(proc_Wt_ref[l], su,
                                          preferred_element_type=jnp.float32)
                                  + proc_b_ref[l])
                HID[bi] = su
                SU[pl.ds(t * NB + bi, 1)] = su[None]
                W5[:, pl.ds(t * NB + bi, 1), :] = w[:, None, :]
                BASE[pl.ds(t * NB + bi, 1)] = base_f
                if bi == 0:
                    ACC = su
                else:
                    ACC = ACC + su

                jl = jnp.sum(jump_Wc_ref[bi] * su, axis=0, keepdims=True) \
                    + jump_b_ref[bi]
                jd = jnp.where(jax.nn.sigmoid(jl) > 0.5, 1.0, 0.0)
                walk = jnp.remainder(ptr + 1.0, P)
                onehot = iotaP == base_i
                destv = jnp.sum(jnp.where(onehot, destT_ref[:, bi:bi + 1],
                                          0.0),
                                axis=0, keepdims=True)
                PTR[bi] = jnp.remainder(jd * destv + (1.0 - jd) * walk, P)[0]
            y_ref[pl.ds(t, 1)] = (jnp.dot(out_Wt_ref[...], ACC * (1.0 / NB),
                                          preferred_element_type=jnp.float32)
                                  + out_b_ref[...])[None]
            return 0
        return step_t

    for k in range(4):
        lax.fori_loop(8 * k, 8 * (k + 1), make_step(k), 0)


@jax.jit
def kernel(x, in_W, in_b, out_W, out_b, proc_W, proc_b, dest, jump_W, jump_b,
           ctx, phase, ptr_init):
    xT = jnp.transpose(x, (1, 2, 0))
    in_Wt = jnp.transpose(in_W)
    out_Wt = jnp.transpose(out_W)
    proc_Wt = jnp.transpose(proc_W, (0, 2, 1))
    destT = jnp.transpose(dest)
    pb = jnp.concatenate(
        [phase, jnp.zeros((NB, D - phase.shape[1]), phase.dtype)], axis=1)
    yT = pl.pallas_call(
        _ring_kernel,
        out_shape=jax.ShapeDtypeStruct((T, 8, B), jnp.float32),
        scratch_shapes=[
            pltpu.VMEM((E, D, B), jnp.float32),
            pltpu.VMEM((5, E, B), jnp.float32),
            pltpu.VMEM((E, B), jnp.float32),
            pltpu.VMEM((NB, B), jnp.float32),
            pltpu.VMEM((NB, D, B), jnp.float32),
        ],
    )(xT, in_Wt, in_b[:, None], out_Wt, out_b[:, None],
      proc_Wt, proc_b[:, :, None], destT, jump_W[:, :, None],
      jump_b[:, None, None], jax.nn.sigmoid(ctx)[:, None, None],
      pb[:, :, None], ptr_init)
    return jnp.transpose(yT, (2, 0, 1))


# speedup vs baseline: 1.2429x; 1.0925x over previous
import jax
import jax.numpy as jnp
from jax import lax
from jax.experimental import pallas as pl
from jax.experimental.pallas import tpu as pltpu

B = 128
T = 32
P = 2048
D = 64
NB = 4
K = 2
TEMP = 8.0
E = T * NB
CHUNK = 32
HALF = P / 2.0


def _ring_kernel(xT_ref, in_Wt_ref, in_b_ref, out_Wt_ref, out_b_ref,
                 proc_Wt_ref, proc_b_ref, partsRT_ref, jump_Wc_ref, jump_b_ref,
                 cs_ref, pb_ref, ptr0_ref,
                 y_ref,
                 SU, W5, BASE, PTR, HID):
    L = proc_Wt_ref.shape[0]
    PTR[...] = ptr0_ref[...]
    HID[...] = jnp.zeros_like(HID)
    SU[...] = jnp.zeros_like(SU)
    W5[...] = jnp.zeros_like(W5)
    BASE[...] = jnp.zeros_like(BASE)

    offs5 = lax.broadcasted_iota(jnp.int32, (5, B), 0).astype(jnp.float32) - K
    iotaC = lax.broadcasted_iota(jnp.int32, (CHUNK, 1), 0).astype(jnp.float32)
    iota16 = lax.broadcasted_iota(jnp.int32, (16, B), 0)
    iota128 = lax.broadcasted_iota(jnp.int32, (128, B), 0)

    def make_step(k):
        def step_t(t, _):
            xt = xT_ref[t]
            inp = jnp.dot(in_Wt_ref[...], xt,
                          preferred_element_type=jnp.float32) + in_b_ref[...]
            for bi in range(NB):
                ptr = PTR[bi][None, :]
                base_i = jnp.clip(jnp.floor(ptr).astype(jnp.int32), 0, P - 1)
                base_f = base_i.astype(jnp.float32)
                idx_f = jnp.mod(base_f + offs5, P)
                delta = jnp.remainder(idx_f - ptr + HALF, P) - HALF
                logits = -(delta * delta) / TEMP
                mx = jnp.max(logits, axis=0, keepdims=True)
                ex = jnp.exp(logits - mx)
                w = ex / jnp.sum(ex, axis=0, keepdims=True)

                context = jnp.zeros((D, B), jnp.float32)
                for ci in range(k + 1):
                    sl = slice(CHUNK * ci, CHUNK * (ci + 1))
                    dd = jnp.remainder(base_f - BASE[sl] + HALF, P) - HALF
                    c = jnp.zeros((CHUNK, B), jnp.float32)
                    for jp in range(5):
                        g = jnp.zeros((CHUNK, B), jnp.float32)
                        for m in range(5):
                            g = g + jnp.where(dd == float(jp - m),
                                              w[m][None, :], 0.0)
                        c = c + W5[jp, sl] * g
                    if ci == k:
                        s_rel = (t * NB + bi - CHUNK * k).astype(jnp.float32)
                        c = c * jnp.where(iotaC < s_rel, 1.0, 0.0)
                    for e in range(CHUNK):
                        context = context + c[e][None, :] * SU[CHUNK * ci + e]

                comb = inp + cs_ref[bi] * context + 0.1 * pb_ref[bi]
                su = jnp.tanh(comb + HID[bi])
                for l in range(L):
                    su = jnp.tanh(jnp.dot(proc_Wt_ref[l], su,
                                          preferred_element_type=jnp.float32)
                                  + proc_b_ref[l])
                HID[bi] = su
                SU[pl.ds(t * NB + bi, 1)] = su[None]
                W5[:, pl.ds(t * NB + bi, 1), :] = w[:, None, :]
                BASE[pl.ds(t * NB + bi, 1)] = base_f
                if bi == 0:
                    ACC = su
                else:
                    ACC = ACC + su

                jl = jnp.sum(jump_Wc_ref[bi] * su, axis=0, keepdims=True) \
                    + jump_b_ref[bi]
                jd = jnp.where(jax.nn.sigmoid(jl) > 0.5, 1.0, 0.0)
                walk = jnp.remainder(ptr + 1.0, P)
                hi = lax.div(base_i, 128)
                lo = base_i - hi * 128
                Mhi = jnp.where(iota16 == hi, 1.0, 0.0)
                dall = jnp.dot(partsRT_ref[bi], Mhi,
                               preferred_element_type=jnp.float32)
                V = (256.0 * dall[0:128] + dall[128:256]) \
                    + jnp.float32(2.0 ** -23) * ((65536.0 * dall[256:384]
                                                  + 256.0 * dall[384:512])
                                                 + dall[512:640])
                Mlo = jnp.where(iota128 == lo, 1.0, 0.0)
                destv = jnp.sum(Mlo * V, axis=0, keepdims=True)
                PTR[bi] = jnp.remainder(jd * destv + (1.0 - jd) * walk, P)[0]
            y_ref[pl.ds(t, 1)] = (jnp.dot(out_Wt_ref[...], ACC * (1.0 / NB),
                                          preferred_element_type=jnp.float32)
                                  + out_b_ref[...])[None]
            return 0
        return step_t

    for k in range(4):
        lax.fori_loop(8 * k, 8 * (k + 1), make_step(k), 0)


@jax.jit
def kernel(x, in_W, in_b, out_W, out_b, proc_W, proc_b, dest, jump_W, jump_b,
           ctx, phase, ptr_init):
    xT = jnp.transpose(x, (1, 2, 0))
    in_Wt = jnp.transpose(in_W)
    out_Wt = jnp.transpose(out_W)
    proc_Wt = jnp.transpose(proc_W, (0, 2, 1))
    vhi = jnp.floor(dest)
    vfrac = dest - vhi
    h1 = jnp.floor(vhi / 256.0)
    h0 = vhi - 256.0 * h1
    f23 = jnp.floor(vfrac * 8388608.0)
    c2 = jnp.floor(f23 / 65536.0)
    r = f23 - 65536.0 * c2
    c1 = jnp.floor(r / 256.0)
    c0 = r - 256.0 * c1
    parts = jnp.stack([h1, h0, c2, c1, c0], axis=1)
    partsRT = parts.reshape(NB, 5, 16, 128).transpose(0, 1, 3, 2) \
                   .reshape(NB, 640, 16)
    pb = jnp.concatenate(
        [phase, jnp.zeros((NB, D - phase.shape[1]), phase.dtype)], axis=1)
    yT = pl.pallas_call(
        _ring_kernel,
        out_shape=jax.ShapeDtypeStruct((T, 8, B), jnp.float32),
        scratch_shapes=[
            pltpu.VMEM((E, D, B), jnp.float32),
            pltpu.VMEM((5, E, B), jnp.float32),
            pltpu.VMEM((E, B), jnp.float32),
            pltpu.VMEM((NB, B), jnp.float32),
            pltpu.VMEM((NB, D, B), jnp.float32),
        ],
    )(xT, in_Wt, in_b[:, None], out_Wt, out_b[:, None],
      proc_Wt, proc_b[:, :, None], partsRT, jump_W[:, :, None],
      jump_b[:, None, None], jax.nn.sigmoid(ctx)[:, None, None],
      pb[:, :, None], ptr_init)
    return jnp.transpose(yT, (2, 0, 1))


# speedup vs baseline: 1.2458x; 1.0023x over previous
import jax
import jax.numpy as jnp
from jax import lax
from jax.experimental import pallas as pl
from jax.experimental.pallas import tpu as pltpu

B = 128
T = 32
P = 2048
D = 64
NB = 4
K = 2
TEMP = 8.0
E = T * NB
CHUNK = 32
HALF = P / 2.0


def _ring_kernel(xT_ref, in_Wt_ref, in_b_ref, out_Wt_ref, out_b_ref,
                 proc_Wt_ref, proc_b_ref, partsRT_ref, jump_Wc_ref, jump_b_ref,
                 cs_ref, pb_ref, ptr0_ref,
                 y_ref,
                 SU, W5, BASE, PTR, HID):
    L = proc_Wt_ref.shape[0]
    PTR[...] = ptr0_ref[...]
    HID[...] = jnp.zeros_like(HID)
    SU[...] = jnp.zeros_like(SU)
    W5[...] = jnp.zeros_like(W5)
    BASE[...] = jnp.zeros_like(BASE)

    offs5 = lax.broadcasted_iota(jnp.int32, (5, B), 0).astype(jnp.float32) - K
    iotaC = lax.broadcasted_iota(jnp.int32, (CHUNK, 1), 0).astype(jnp.float32)
    iota16 = lax.broadcasted_iota(jnp.int32, (16, B), 0)
    iota128 = lax.broadcasted_iota(jnp.int32, (128, B), 0)

    def make_step(k):
        def step_t(t, _):
            xt = xT_ref[t]
            inp = jnp.dot(in_Wt_ref[...], xt,
                          preferred_element_type=jnp.float32) + in_b_ref[...]
            for bi in range(NB):
                ptr = PTR[bi][None, :]
                base_i = jnp.clip(jnp.floor(ptr).astype(jnp.int32), 0, P - 1)
                base_f = base_i.astype(jnp.float32)
                idx_f = jnp.mod(base_f + offs5, P)
                delta = jnp.remainder(idx_f - ptr + HALF, P) - HALF
                logits = -(delta * delta) / TEMP
                mx = jnp.max(logits, axis=0, keepdims=True)
                ex = jnp.exp(logits - mx)
                w = ex / jnp.sum(ex, axis=0, keepdims=True)

                acc0 = jnp.zeros((D, B), jnp.float32)
                acc1 = jnp.zeros((D, B), jnp.float32)
                acc2 = jnp.zeros((D, B), jnp.float32)
                acc3 = jnp.zeros((D, B), jnp.float32)
                for ci in range(k + 1):
                    sl = slice(CHUNK * ci, CHUNK * (ci + 1))
                    dd = jnp.remainder(base_f - BASE[sl] + HALF, P) - HALF
                    c = jnp.zeros((CHUNK, B), jnp.float32)
                    for jp in range(5):
                        g = jnp.zeros((CHUNK, B), jnp.float32)
                        for m in range(5):
                            g = g + jnp.where(dd == float(jp - m),
                                              w[m][None, :], 0.0)
                        c = c + W5[jp, sl] * g
                    if ci == k:
                        s_rel = (t * NB + bi - CHUNK * k).astype(jnp.float32)
                        c = c * jnp.where(iotaC < s_rel, 1.0, 0.0)
                    for e in range(0, CHUNK, 4):
                        base_e = CHUNK * ci + e
                        acc0 = acc0 + c[e][None, :] * SU[base_e]
                        acc1 = acc1 + c[e + 1][None, :] * SU[base_e + 1]
                        acc2 = acc2 + c[e + 2][None, :] * SU[base_e + 2]
                        acc3 = acc3 + c[e + 3][None, :] * SU[base_e + 3]

                context = (acc0 + acc1) + (acc2 + acc3)
                comb = inp + cs_ref[bi] * context + 0.1 * pb_ref[bi]
                su = jnp.tanh(comb + HID[bi])
                for l in range(L):
                    su = jnp.tanh(jnp.dot(proc_Wt_ref[l], su,
                                          preferred_element_type=jnp.float32)
                                  + proc_b_ref[l])
                HID[bi] = su
                SU[pl.ds(t * NB + bi, 1)] = su[None]
                W5[:, pl.ds(t * NB + bi, 1), :] = w[:, None, :]
                BASE[pl.ds(t * NB + bi, 1)] = base_f
                if bi == 0:
                    ACC = su
                else:
                    ACC = ACC + su

                jl = jnp.sum(jump_Wc_ref[bi] * su, axis=0, keepdims=True) \
                    + jump_b_ref[bi]
                jd = jnp.where(jax.nn.sigmoid(jl) > 0.5, 1.0, 0.0)
                walk = jnp.remainder(ptr + 1.0, P)
                hi = lax.div(base_i, 128)
                lo = base_i - hi * 128
                Mhi = jnp.where(iota16 == hi, 1.0, 0.0)
                dall = jnp.dot(partsRT_ref[bi], Mhi,
                               preferred_element_type=jnp.float32)
                V = (256.0 * dall[0:128] + dall[128:256]) \
                    + jnp.float32(2.0 ** -23) * ((65536.0 * dall[256:384]
                                                  + 256.0 * dall[384:512])
                                                 + dall[512:640])
                Mlo = jnp.where(iota128 == lo, 1.0, 0.0)
                destv = jnp.sum(Mlo * V, axis=0, keepdims=True)
                PTR[bi] = jnp.remainder(jd * destv + (1.0 - jd) * walk, P)[0]
            y_ref[pl.ds(t, 1)] = (jnp.dot(out_Wt_ref[...], ACC * (1.0 / NB),
                                          preferred_element_type=jnp.float32)
                                  + out_b_ref[...])[None]
            return 0
        return step_t

    for k in range(4):
        lax.fori_loop(8 * k, 8 * (k + 1), make_step(k), 0)


@jax.jit
def kernel(x, in_W, in_b, out_W, out_b, proc_W, proc_b, dest, jump_W, jump_b,
           ctx, phase, ptr_init):
    xT = jnp.transpose(x, (1, 2, 0))
    in_Wt = jnp.transpose(in_W)
    out_Wt = jnp.transpose(out_W)
    proc_Wt = jnp.transpose(proc_W, (0, 2, 1))
    vhi = jnp.floor(dest)
    vfrac = dest - vhi
    h1 = jnp.floor(vhi / 256.0)
    h0 = vhi - 256.0 * h1
    f23 = jnp.floor(vfrac * 8388608.0)
    c2 = jnp.floor(f23 / 65536.0)
    r = f23 - 65536.0 * c2
    c1 = jnp.floor(r / 256.0)
    c0 = r - 256.0 * c1
    parts = jnp.stack([h1, h0, c2, c1, c0], axis=1)
    partsRT = parts.reshape(NB, 5, 16, 128).transpose(0, 1, 3, 2) \
                   .reshape(NB, 640, 16)
    pb = jnp.concatenate(
        [phase, jnp.zeros((NB, D - phase.shape[1]), phase.dtype)], axis=1)
    yT = pl.pallas_call(
        _ring_kernel,
        out_shape=jax.ShapeDtypeStruct((T, 8, B), jnp.float32),
        scratch_shapes=[
            pltpu.VMEM((E, D, B), jnp.float32),
            pltpu.VMEM((5, E, B), jnp.float32),
            pltpu.VMEM((E, B), jnp.float32),
            pltpu.VMEM((NB, B), jnp.float32),
            pltpu.VMEM((NB, D, B), jnp.float32),
        ],
    )(xT, in_Wt, in_b[:, None], out_Wt, out_b[:, None],
      proc_Wt, proc_b[:, :, None], partsRT, jump_W[:, :, None],
      jump_b[:, None, None], jax.nn.sigmoid(ctx)[:, None, None],
      pb[:, :, None], ptr_init)
    return jnp.transpose(yT, (2, 0, 1))


# speedup vs baseline: 1.4119x; 1.1333x over previous
import jax
import jax.numpy as jnp
from jax import lax
from jax.experimental import pallas as pl
from jax.experimental.pallas import tpu as pltpu

B = 128
T = 32
P = 2048
D = 64
NB = 4
K = 2
TEMP = 8.0
E = T * NB
CHUNK = 32
HALF = P / 2.0


def _ring_kernel(xT_ref, in_Wt_ref, in_b_ref, out_Wt_ref, out_b_ref,
                 proc_Wt_ref, proc_b_ref, partsRT_ref, jump_Wc_ref, jump_b_ref,
                 cs_ref, pb_ref, ptr0_ref,
                 y_ref,
                 SU, W5, BASE, PTR, HID):
    L = proc_Wt_ref.shape[0]
    PTR[...] = ptr0_ref[...]
    HID[...] = jnp.zeros_like(HID)
    SU[...] = jnp.zeros_like(SU)
    W5[...] = jnp.zeros_like(W5)
    BASE[...] = jnp.zeros_like(BASE)

    offs5 = lax.broadcasted_iota(jnp.int32, (5, B), 0).astype(jnp.float32) - K
    iotaC = lax.broadcasted_iota(jnp.int32, (CHUNK, 1), 0).astype(jnp.float32)
    iota16 = lax.broadcasted_iota(jnp.int32, (16, B), 0)
    iota128 = lax.broadcasted_iota(jnp.int32, (128, B), 0)

    def make_step(k):
        def step_t(t, _):
            xt = xT_ref[t]
            inp = jnp.dot(in_Wt_ref[...], xt,
                          preferred_element_type=jnp.float32) + in_b_ref[...]

            ptr_l, base_i_l, base_f_l, w_l = [], [], [], []
            for bi in range(NB):
                ptr = PTR[bi][None, :]
                base_i = jnp.clip(jnp.floor(ptr).astype(jnp.int32), 0, P - 1)
                base_f = base_i.astype(jnp.float32)
                idx_f = jnp.mod(base_f + offs5, P)
                delta = jnp.remainder(idx_f - ptr + HALF, P) - HALF
                logits = -(delta * delta) / TEMP
                mx = jnp.max(logits, axis=0, keepdims=True)
                ex = jnp.exp(logits - mx)
                w = ex / jnp.sum(ex, axis=0, keepdims=True)
                ptr_l.append(ptr)
                base_i_l.append(base_i)
                base_f_l.append(base_f)
                w_l.append(w)

            c_l = [[None] * (k + 1) for _ in range(NB)]
            for bi in range(NB):
                for ci in range(k + 1):
                    sl = slice(CHUNK * ci, CHUNK * (ci + 1))
                    dd = jnp.remainder(base_f_l[bi] - BASE[sl] + HALF,
                                       P) - HALF
                    c = jnp.zeros((CHUNK, B), jnp.float32)
                    for jp in range(5):
                        g = jnp.zeros((CHUNK, B), jnp.float32)
                        for m in range(5):
                            g = g + jnp.where(dd == float(jp - m),
                                              w_l[bi][m][None, :], 0.0)
                        c = c + W5[jp, sl] * g
                    if ci == k:
                        s_rel = (t * NB - CHUNK * k).astype(jnp.float32)
                        c = c * jnp.where(iotaC < s_rel, 1.0, 0.0)
                    c_l[bi][ci] = c

            acc = [jnp.zeros((D, B), jnp.float32) for _ in range(NB)]
            for ci in range(k + 1):
                for e in range(CHUNK):
                    su_e = SU[CHUNK * ci + e]
                    for bi in range(NB):
                        acc[bi] = acc[bi] + c_l[bi][ci][e][None, :] * su_e

            su_t = []
            for bi in range(NB):
                ctxv = acc[bi]
                for j in range(bi):
                    ddp = jnp.remainder(base_f_l[bi] - base_f_l[j] + HALF,
                                        P) - HALF
                    cp = jnp.zeros((1, B), jnp.float32)
                    for jp in range(5):
                        g = jnp.zeros((1, B), jnp.float32)
                        for m in range(5):
                            g = g + jnp.where(ddp == float(jp - m),
                                              w_l[bi][m][None, :], 0.0)
                        cp = cp + w_l[j][jp][None, :] * g
                    ctxv = ctxv + cp * su_t[j]
                comb = inp + cs_ref[bi] * ctxv + 0.1 * pb_ref[bi]
                su = jnp.tanh(comb + HID[bi])
                for l in range(L):
                    su = jnp.tanh(jnp.dot(proc_Wt_ref[l], su,
                                          preferred_element_type=jnp.float32)
                                  + proc_b_ref[l])
                HID[bi] = su
                SU[pl.ds(t * NB + bi, 1)] = su[None]
                W5[:, pl.ds(t * NB + bi, 1), :] = w_l[bi][:, None, :]
                BASE[pl.ds(t * NB + bi, 1)] = base_f_l[bi]
                su_t.append(su)
                if bi == 0:
                    ACC = su
                else:
                    ACC = ACC + su

                jl = jnp.sum(jump_Wc_ref[bi] * su, axis=0, keepdims=True) \
                    + jump_b_ref[bi]
                jd = jnp.where(jax.nn.sigmoid(jl) > 0.5, 1.0, 0.0)
                walk = jnp.remainder(ptr_l[bi] + 1.0, P)
                hi = lax.div(base_i_l[bi], 128)
                lo = base_i_l[bi] - hi * 128
                Mhi = jnp.where(iota16 == hi, 1.0, 0.0)
                dall = jnp.dot(partsRT_ref[bi], Mhi,
                               preferred_element_type=jnp.float32)
                V = (256.0 * dall[0:128] + dall[128:256]) \
                    + jnp.float32(2.0 ** -23) * ((65536.0 * dall[256:384]
                                                  + 256.0 * dall[384:512])
                                                 + dall[512:640])
                Mlo = jnp.where(iota128 == lo, 1.0, 0.0)
                destv = jnp.sum(Mlo * V, axis=0, keepdims=True)
                PTR[bi] = jnp.remainder(jd * destv
                                        + (1.0 - jd) * walk, P)[0]
            y_ref[pl.ds(t, 1)] = (jnp.dot(out_Wt_ref[...], ACC * (1.0 / NB),
                                          preferred_element_type=jnp.float32)
                                  + out_b_ref[...])[None]
            return 0
        return step_t

    for k in range(4):
        lax.fori_loop(8 * k, 8 * (k + 1), make_step(k), 0)


@jax.jit
def kernel(x, in_W, in_b, out_W, out_b, proc_W, proc_b, dest, jump_W, jump_b,
           ctx, phase, ptr_init):
    xT = jnp.transpose(x, (1, 2, 0))
    in_Wt = jnp.transpose(in_W)
    out_Wt = jnp.transpose(out_W)
    proc_Wt = jnp.transpose(proc_W, (0, 2, 1))
    vhi = jnp.floor(dest)
    vfrac = dest - vhi
    h1 = jnp.floor(vhi / 256.0)
    h0 = vhi - 256.0 * h1
    f23 = jnp.floor(vfrac * 8388608.0)
    c2 = jnp.floor(f23 / 65536.0)
    r = f23 - 65536.0 * c2
    c1 = jnp.floor(r / 256.0)
    c0 = r - 256.0 * c1
    parts = jnp.stack([h1, h0, c2, c1, c0], axis=1)
    partsRT = parts.reshape(NB, 5, 16, 128).transpose(0, 1, 3, 2) \
                   .reshape(NB, 640, 16)
    pb = jnp.concatenate(
        [phase, jnp.zeros((NB, D - phase.shape[1]), phase.dtype)], axis=1)
    yT = pl.pallas_call(
        _ring_kernel,
        out_shape=jax.ShapeDtypeStruct((T, 8, B), jnp.float32),
        scratch_shapes=[
            pltpu.VMEM((E, D, B), jnp.float32),
            pltpu.VMEM((5, E, B), jnp.float32),
            pltpu.VMEM((E, B), jnp.float32),
            pltpu.VMEM((NB, B), jnp.float32),
            pltpu.VMEM((NB, D, B), jnp.float32),
        ],
    )(xT, in_Wt, in_b[:, None], out_Wt, out_b[:, None],
      proc_Wt, proc_b[:, :, None], partsRT, jump_W[:, :, None],
      jump_b[:, None, None], jax.nn.sigmoid(ctx)[:, None, None],
      pb[:, :, None], ptr_init)
    return jnp.transpose(yT, (2, 0, 1))


# speedup vs baseline: 1.5133x; 1.0719x over previous
import jax
import jax.numpy as jnp
from jax import lax
from jax.experimental import pallas as pl
from jax.experimental.pallas import tpu as pltpu

B = 128
T = 32
P = 2048
D = 64
NB = 4
K = 2
TEMP = 8.0
E = T * NB
CHUNK = 16
HALF = P / 2.0


def _ring_kernel(xT_ref, in_Wt_ref, in_b_ref, out_Wt_ref, out_b_ref,
                 proc_Wt_ref, proc_b_ref, partsRT_ref, jump_Wc_ref, jump_b_ref,
                 cs_ref, pb_ref, ptr0_ref,
                 y_ref,
                 SU, W5, BASE, PTR, HID):
    L = proc_Wt_ref.shape[0]
    PTR[...] = ptr0_ref[...]
    HID[...] = jnp.zeros_like(HID)
    SU[...] = jnp.zeros_like(SU)
    W5[...] = jnp.zeros_like(W5)
    BASE[...] = jnp.zeros_like(BASE)

    offs5 = lax.broadcasted_iota(jnp.int32, (5, B), 0).astype(jnp.float32) - K
    iotaC = lax.broadcasted_iota(jnp.int32, (CHUNK, 1), 0).astype(jnp.float32)
    iota16 = lax.broadcasted_iota(jnp.int32, (16, B), 0)
    iota128 = lax.broadcasted_iota(jnp.int32, (128, B), 0)

    def make_step(k):
        def step_t(t, _):
            xt = xT_ref[t]
            inp = jnp.dot(in_Wt_ref[...], xt,
                          preferred_element_type=jnp.float32) + in_b_ref[...]

            ptr_l, base_i_l, base_f_l, w_l = [], [], [], []
            for bi in range(NB):
                ptr = PTR[bi][None, :]
                base_i = jnp.clip(jnp.floor(ptr).astype(jnp.int32), 0, P - 1)
                base_f = base_i.astype(jnp.float32)
                idx_f = jnp.mod(base_f + offs5, P)
                delta = jnp.remainder(idx_f - ptr + HALF, P) - HALF
                logits = -(delta * delta) / TEMP
                mx = jnp.max(logits, axis=0, keepdims=True)
                ex = jnp.exp(logits - mx)
                w = ex / jnp.sum(ex, axis=0, keepdims=True)
                ptr_l.append(ptr)
                base_i_l.append(base_i)
                base_f_l.append(base_f)
                w_l.append(w)

            c_l = [[None] * (k + 1) for _ in range(NB)]
            for bi in range(NB):
                for ci in range(k + 1):
                    sl = slice(CHUNK * ci, CHUNK * (ci + 1))
                    dd = jnp.remainder(base_f_l[bi] - BASE[sl] + HALF,
                                       P) - HALF
                    c = jnp.zeros((CHUNK, B), jnp.float32)
                    for kk in range(-4, 5):
                        r = None
                        for jp in range(max(0, kk), min(4, 4 + kk) + 1):
                            term = W5[jp, sl] * w_l[bi][jp - kk][None, :]
                            r = term if r is None else r + term
                        c = jnp.where(dd == float(kk), r, c)
                    if ci == k:
                        s_rel = (t * NB - CHUNK * k).astype(jnp.float32)
                        c = c * jnp.where(iotaC < s_rel, 1.0, 0.0)
                    c_l[bi][ci] = c

            acc = [jnp.zeros((D, B), jnp.float32) for _ in range(NB)]
            for ci in range(k + 1):
                for e in range(CHUNK):
                    su_e = SU[CHUNK * ci + e]
                    for bi in range(NB):
                        acc[bi] = acc[bi] + c_l[bi][ci][e][None, :] * su_e

            su_t = []
            for bi in range(NB):
                ctxv = acc[bi]
                for j in range(bi):
                    ddp = jnp.remainder(base_f_l[bi] - base_f_l[j] + HALF,
                                        P) - HALF
                    cp = jnp.zeros((1, B), jnp.float32)
                    for kk in range(-4, 5):
                        r = None
                        for jp in range(max(0, kk), min(4, 4 + kk) + 1):
                            term = w_l[j][jp][None, :] * w_l[bi][jp - kk][None, :]
                            r = term if r is None else r + term
                        cp = jnp.where(ddp == float(kk), r, cp)
                    ctxv = ctxv + cp * su_t[j]
                comb = inp + cs_ref[bi] * ctxv + 0.1 * pb_ref[bi]
                su = jnp.tanh(comb + HID[bi])
                for l in range(L):
                    su = jnp.tanh(jnp.dot(proc_Wt_ref[l], su,
                                          preferred_element_type=jnp.float32)
                                  + proc_b_ref[l])
                HID[bi] = su
                SU[pl.ds(t * NB + bi, 1)] = su[None]
                W5[:, pl.ds(t * NB + bi, 1), :] = w_l[bi][:, None, :]
                BASE[pl.ds(t * NB + bi, 1)] = base_f_l[bi]
                su_t.append(su)
                if bi == 0:
                    ACC = su
                else:
                    ACC = ACC + su

                jl = jnp.sum(jump_Wc_ref[bi] * su, axis=0, keepdims=True) \
                    + jump_b_ref[bi]
                jd = jnp.where(jax.nn.sigmoid(jl) > 0.5, 1.0, 0.0)
                walk = jnp.remainder(ptr_l[bi] + 1.0, P)
                hi = lax.div(base_i_l[bi], 128)
                lo = base_i_l[bi] - hi * 128
                Mhi = jnp.where(iota16 == hi, 1.0, 0.0)
                dall = jnp.dot(partsRT_ref[bi], Mhi,
                               preferred_element_type=jnp.float32)
                V = (256.0 * dall[0:128] + dall[128:256]) \
                    + jnp.float32(2.0 ** -23) * ((65536.0 * dall[256:384]
                                                  + 256.0 * dall[384:512])
                                                 + dall[512:640])
                Mlo = jnp.where(iota128 == lo, 1.0, 0.0)
                destv = jnp.sum(Mlo * V, axis=0, keepdims=True)
                PTR[bi] = jnp.remainder(jd * destv
                                        + (1.0 - jd) * walk, P)[0]
            y_ref[pl.ds(t, 1)] = (jnp.dot(out_Wt_ref[...], ACC * (1.0 / NB),
                                          preferred_element_type=jnp.float32)
                                  + out_b_ref[...])[None]
            return 0
        return step_t

    for k in range(8):
        lax.fori_loop(4 * k, 4 * (k + 1), make_step(k), 0)


@jax.jit
def kernel(x, in_W, in_b, out_W, out_b, proc_W, proc_b, dest, jump_W, jump_b,
           ctx, phase, ptr_init):
    xT = jnp.transpose(x, (1, 2, 0))
    in_Wt = jnp.transpose(in_W)
    out_Wt = jnp.transpose(out_W)
    proc_Wt = jnp.transpose(proc_W, (0, 2, 1))
    vhi = jnp.floor(dest)
    vfrac = dest - vhi
    h1 = jnp.floor(vhi / 256.0)
    h0 = vhi - 256.0 * h1
    f23 = jnp.floor(vfrac * 8388608.0)
    c2 = jnp.floor(f23 / 65536.0)
    r = f23 - 65536.0 * c2
    c1 = jnp.floor(r / 256.0)
    c0 = r - 256.0 * c1
    parts = jnp.stack([h1, h0, c2, c1, c0], axis=1)
    partsRT = parts.reshape(NB, 5, 16, 128).transpose(0, 1, 3, 2) \
                   .reshape(NB, 640, 16)
    pb = jnp.concatenate(
        [phase, jnp.zeros((NB, D - phase.shape[1]), phase.dtype)], axis=1)
    yT = pl.pallas_call(
        _ring_kernel,
        out_shape=jax.ShapeDtypeStruct((T, 8, B), jnp.float32),
        scratch_shapes=[
            pltpu.VMEM((E, D, B), jnp.float32),
            pltpu.VMEM((5, E, B), jnp.float32),
            pltpu.VMEM((E, B), jnp.float32),
            pltpu.VMEM((NB, B), jnp.float32),
            pltpu.VMEM((NB, D, B), jnp.float32),
        ],
    )(xT, in_Wt, in_b[:, None], out_Wt, out_b[:, None],
      proc_Wt, proc_b[:, :, None], partsRT, jump_W[:, :, None],
      jump_b[:, None, None], jax.nn.sigmoid(ctx)[:, None, None],
      pb[:, :, None], ptr_init)
    return jnp.transpose(yT, (2, 0, 1))
